# baseline (speedup 1.0000x reference)
import functools

import jax
import jax.numpy as jnp
from jax import lax
from jax.experimental import pallas as pl
from jax.experimental.pallas import tpu as pltpu
from jax.experimental.pallas import tpu_sc as plsc

VOCAB = 1000000
EMBED_DIM = 32
BATCH = 4096
MAX_LEN = 200

_INFO = plsc.get_sparse_core_info()
_NC = _INFO.num_cores
_NS = _INFO.num_subcores
_NW = _NC * _NS

_BB = BATCH // _NW
_DT = EMBED_DIM // 8
_PAIRS = MAX_LEN // 2


def _make_kernel():
    mesh = plsc.VectorSubcoreMesh(core_axis_name="c", subcore_axis_name="s")

    @functools.partial(
        pl.kernel,
        mesh=mesh,
        compiler_params=pltpu.CompilerParams(
            use_tc_tiling_on_sc=False, needs_layout_passes=False
        ),
        out_type=jax.ShapeDtypeStruct((MAX_LEN, _DT, _NW, 8, _BB), jnp.float32),
        scratch_types=[
            pltpu.VMEM((MAX_LEN, _BB), jnp.int32),
            pltpu.VMEM((_BB, EMBED_DIM), jnp.float32),
            pltpu.VMEM((_BB, EMBED_DIM), jnp.float32),
            pltpu.VMEM((_DT, 8, _BB), jnp.float32),
            pltpu.VMEM((_DT, 8, _BB), jnp.float32),
            pltpu.SemaphoreType.DMA,
            pltpu.SemaphoreType.DMA,
            pltpu.SemaphoreType.DMA,
            pltpu.SemaphoreType.DMA,
        ],
    )
    def emb_kernel(idx_hbm, table_hbm, out_hbm, idx_v, buf0, buf1, bt0, bt1,
                   sg0, sg1, ss0, ss1):
        wid = lax.axis_index("s") * _NC + lax.axis_index("c")
        pltpu.sync_copy(idx_hbm.at[wid], idx_v)
        lanes = lax.iota(jnp.int32, 16)
        dts = [(d + lanes) >> 3 for d in (0, 16)]
        dds = [(d + lanes) & 7 for d in (0, 16)]

        def fire(l, buf, sem):
            pltpu.async_copy(table_hbm.at[idx_v.at[l]], buf, sem)

        def drain_gather(buf, sem):
            pltpu.make_async_copy(table_hbm.at[pl.ds(0, _BB)], buf, sem).wait()

        def transpose(buf, bt):
            def tbody(bb, _):
                bbs = jnp.full((16,), bb, jnp.int32)
                for h, d0 in enumerate((0, 16)):
                    vals = buf[bb, pl.ds(d0, 16)]
                    plsc.store_scatter(bt, [dts[h], dds[h], bbs], vals)
                return 0

            lax.fori_loop(0, _BB, tbody, 0)

        def store_start(bt, l, sem):
            pltpu.async_copy(bt, out_hbm.at[l, :, wid, :, :], sem)

        def store_wait(bt, sem):
            pltpu.make_async_copy(bt, out_hbm.at[0, :, 0, :, :], sem).wait()

        def step(l, buf, bt, sg, ss, fire_next):
            drain_gather(buf, sg)
            transpose(buf, bt)
            store_start(bt, l, ss)
            if fire_next:
                fire(l + 2, buf, sg)

        fire(0, buf0, sg0)
        fire(1, buf1, sg1)
        step(0, buf0, bt0, sg0, ss0, True)
        step(1, buf1, bt1, sg1, ss1, True)

        def pair(p, _):
            l0 = 2 * p
            store_wait(bt0, ss0)
            step(l0, buf0, bt0, sg0, ss0, True)
            store_wait(bt1, ss1)
            step(l0 + 1, buf1, bt1, sg1, ss1, True)
            return 0

        lax.fori_loop(1, _PAIRS - 1, pair, 0)

        l0 = MAX_LEN - 2
        store_wait(bt0, ss0)
        step(l0, buf0, bt0, sg0, ss0, False)
        store_wait(bt1, ss1)
        step(l0 + 1, buf1, bt1, sg1, ss1, False)
        store_wait(bt0, ss0)
        store_wait(bt1, ss1)

    return emb_kernel


_EMB_KERNEL = _make_kernel()


@jax.jit
def kernel(inputs, table):
    idx = inputs.astype(jnp.int32).T.reshape(MAX_LEN, _NW, _BB).transpose(1, 0, 2)
    out5 = _EMB_KERNEL(idx, table)
    return out5.transpose(2, 4, 0, 1, 3).reshape(BATCH, MAX_LEN, EMBED_DIM)
